# baseline (device time: 268511 ns/iter reference)
import jax
import jax.numpy as jnp
from jax import lax
from jax.experimental import pallas as pl
from jax.experimental.pallas import tpu as pltpu

N_DEV = 32
B_LOC = 2
SQ = 128
SKV = 128
HQ = 128
DH = 64
D_MODEL = 512
H_LOC = HQ // N_DEV
DF_LOC = H_LOC * DH
TOK = B_LOC * SQ


def kernel(x, Wq, K_ext, V_ext, Wo):
    my_i = lax.axis_index("i")

    x2d = x.reshape(TOK, D_MODEL).astype(jnp.bfloat16)
    wio = jnp.stack([Wq.astype(jnp.bfloat16), Wo.T.astype(jnp.bfloat16)])

    k_loc = lax.dynamic_slice_in_dim(K_ext, my_i * B_LOC, B_LOC, axis=0)
    v_loc = lax.dynamic_slice_in_dim(V_ext, my_i * B_LOC, B_LOC, axis=0)
    kh = k_loc.transpose(2, 0, 1, 3).reshape(HQ * B_LOC, SKV, DH)
    vh = v_loc.transpose(2, 0, 1, 3).reshape(HQ * B_LOC, SKV, DH)
    kh = kh.astype(jnp.bfloat16)
    vh = vh.astype(jnp.bfloat16)

    def body(x_ref, wio_ref, k_ref, v_ref, out_ref, comm_ref, ctx_ref,
             send_sems, recv_sems):
        i = lax.axis_index("i")
        left = lax.rem(i - 1 + N_DEV, N_DEV)
        right = lax.rem(i + 1, N_DEV)

        barrier_sem = pltpu.get_barrier_semaphore()
        pl.semaphore_signal(barrier_sem, inc=1, device_id=(left,),
                            device_id_type=pl.DeviceIdType.MESH)
        pl.semaphore_signal(barrier_sem, inc=1, device_id=(right,),
                            device_id_type=pl.DeviceIdType.MESH)
        pl.semaphore_wait(barrier_sem, 2)

        comm_ref[0] = wio_ref[...]
        out_ref[...] = jnp.zeros_like(out_ref)

        def hop(h, carry):
            slot = lax.rem(h, 2)
            nslot = lax.rem(h + 1, 2)
            j = lax.rem(i - h + N_DEV, N_DEV)

            rdma = pltpu.make_async_remote_copy(
                src_ref=comm_ref.at[slot],
                dst_ref=comm_ref.at[nslot],
                send_sem=send_sems.at[slot],
                recv_sem=recv_sems.at[nslot],
                device_id=(right,),
                device_id_type=pl.DeviceIdType.MESH,
            )

            @pl.when(h < N_DEV - 1)
            def _():
                rdma.start()

            wq = comm_ref[slot, 0]
            woT = comm_ref[slot, 1]

            q_all = lax.dot_general(
                x_ref[...], wq, (((1,), (0,)), ((), ())),
                preferred_element_type=jnp.float32)

            for b in range(B_LOC):
                for hh in range(H_LOC):
                    q = q_all[b * SQ:(b + 1) * SQ,
                              hh * DH:(hh + 1) * DH].astype(jnp.bfloat16)
                    idx = (j * H_LOC + hh) * B_LOC + b
                    k = k_ref[idx]
                    v = v_ref[idx]
                    s = lax.dot_general(
                        q, k, (((1,), (1,)), ((), ())),
                        preferred_element_type=jnp.float32) * 0.125
                    m = jnp.max(s, axis=1, keepdims=True)
                    e = jnp.exp(s - m)
                    w = (e / jnp.sum(e, axis=1, keepdims=True)
                         ).astype(jnp.bfloat16)
                    ctx = lax.dot_general(
                        w, v, (((1,), (0,)), ((), ())),
                        preferred_element_type=jnp.float32)
                    ctx_ref[b * SQ:(b + 1) * SQ,
                            hh * DH:(hh + 1) * DH] = ctx.astype(jnp.bfloat16)

            partial = lax.dot_general(
                ctx_ref[...], woT, (((1,), (1,)), ((), ())),
                preferred_element_type=jnp.float32)
            out_ref[...] += partial.reshape(B_LOC, SQ, D_MODEL)

            @pl.when(h < N_DEV - 1)
            def _():
                rdma.wait()

            return carry

        lax.fori_loop(0, N_DEV, hop, 0)

    return pl.pallas_call(
        body,
        out_shape=jax.ShapeDtypeStruct((B_LOC, SQ, D_MODEL), jnp.float32),
        in_specs=[
            pl.BlockSpec(memory_space=pltpu.VMEM),
            pl.BlockSpec(memory_space=pltpu.VMEM),
            pl.BlockSpec(memory_space=pltpu.VMEM),
            pl.BlockSpec(memory_space=pltpu.VMEM),
        ],
        out_specs=pl.BlockSpec(memory_space=pltpu.VMEM),
        scratch_shapes=[
            pltpu.VMEM((2, 2, D_MODEL, DF_LOC), jnp.bfloat16),
            pltpu.VMEM((TOK, DF_LOC), jnp.bfloat16),
            pltpu.SemaphoreType.DMA((2,)),
            pltpu.SemaphoreType.DMA((2,)),
        ],
        compiler_params=pltpu.CompilerParams(collective_id=0),
    )(x2d, wio, kh, vh)


# device time: 242276 ns/iter; 1.1083x vs baseline; 1.1083x over previous
import jax
import jax.numpy as jnp
from jax import lax
from jax.experimental import pallas as pl
from jax.experimental.pallas import tpu as pltpu

N_DEV = 32
B_LOC = 2
SQ = 128
SKV = 128
HQ = 128
DH = 64
D_MODEL = 512
H_LOC = HQ // N_DEV
DF_LOC = H_LOC * DH
TOK = B_LOC * SQ
N_STEP = N_DEV // 2


def kernel(x, Wq, K_ext, V_ext, Wo):
    my_i = lax.axis_index("i")

    x2d = x.reshape(TOK, D_MODEL).astype(jnp.bfloat16)
    wio = jnp.stack([Wq.astype(jnp.bfloat16), Wo.T.astype(jnp.bfloat16)])

    k_loc = lax.dynamic_slice_in_dim(K_ext, my_i * B_LOC, B_LOC, axis=0)
    v_loc = lax.dynamic_slice_in_dim(V_ext, my_i * B_LOC, B_LOC, axis=0)
    kh = k_loc.transpose(2, 0, 1, 3).reshape(HQ * B_LOC, SKV, DH)
    vh = v_loc.transpose(2, 0, 1, 3).reshape(HQ * B_LOC, SKV, DH)
    kh = kh.astype(jnp.bfloat16)
    vh = vh.astype(jnp.bfloat16)

    def body(x_ref, wio_ref, k_ref, v_ref, out_ref,
             cw_ref, ccw_ref, ctx_ref,
             cw_send, cw_recv, ccw_send, ccw_recv):
        i = lax.axis_index("i")
        left = lax.rem(i - 1 + N_DEV, N_DEV)
        right = lax.rem(i + 1, N_DEV)

        barrier_sem = pltpu.get_barrier_semaphore()
        pl.semaphore_signal(barrier_sem, inc=1, device_id=(left,),
                            device_id_type=pl.DeviceIdType.MESH)
        pl.semaphore_signal(barrier_sem, inc=1, device_id=(right,),
                            device_id_type=pl.DeviceIdType.MESH)
        pl.semaphore_wait(barrier_sem, 2)

        cw_ref[0] = wio_ref[...]
        ccw_ref[0] = wio_ref[...]
        out_ref[...] = jnp.zeros_like(out_ref)

        def compute_group(j, comm_ref, slot):
            wq = comm_ref[slot, 0]
            woT = comm_ref[slot, 1]

            q_all = lax.dot_general(
                x_ref[...], wq, (((1,), (0,)), ((), ())),
                preferred_element_type=jnp.float32)

            for b in range(B_LOC):
                for hh in range(H_LOC):
                    q = q_all[b * SQ:(b + 1) * SQ,
                              hh * DH:(hh + 1) * DH].astype(jnp.bfloat16)
                    idx = (j * H_LOC + hh) * B_LOC + b
                    k = k_ref[idx]
                    v = v_ref[idx]
                    s = lax.dot_general(
                        q, k, (((1,), (1,)), ((), ())),
                        preferred_element_type=jnp.float32) * 0.125
                    m = jnp.max(s, axis=1, keepdims=True)
                    e = jnp.exp(s - m)
                    w = (e / jnp.sum(e, axis=1, keepdims=True)
                         ).astype(jnp.bfloat16)
                    ctx = lax.dot_general(
                        w, v, (((1,), (0,)), ((), ())),
                        preferred_element_type=jnp.float32)
                    ctx_ref[b * SQ:(b + 1) * SQ,
                            hh * DH:(hh + 1) * DH] = ctx.astype(jnp.bfloat16)

            partial = lax.dot_general(
                ctx_ref[...], woT, (((1,), (1,)), ((), ())),
                preferred_element_type=jnp.float32)
            out_ref[...] += partial.reshape(B_LOC, SQ, D_MODEL)

        def step(s, carry):
            slot = lax.rem(s, 2)
            nslot = lax.rem(s + 1, 2)

            rdma_cw = pltpu.make_async_remote_copy(
                src_ref=cw_ref.at[slot],
                dst_ref=cw_ref.at[nslot],
                send_sem=cw_send.at[slot],
                recv_sem=cw_recv.at[nslot],
                device_id=(right,),
                device_id_type=pl.DeviceIdType.MESH,
            )
            rdma_ccw = pltpu.make_async_remote_copy(
                src_ref=ccw_ref.at[slot],
                dst_ref=ccw_ref.at[nslot],
                send_sem=ccw_send.at[slot],
                recv_sem=ccw_recv.at[nslot],
                device_id=(left,),
                device_id_type=pl.DeviceIdType.MESH,
            )

            @pl.when(s < N_STEP)
            def _():
                rdma_cw.start()

            @pl.when(s < N_STEP - 1)
            def _():
                rdma_ccw.start()

            compute_group(lax.rem(i - s + N_DEV, N_DEV), cw_ref, slot)

            @pl.when(jnp.logical_and(s >= 1, s < N_STEP))
            def _():
                compute_group(lax.rem(i + s, N_DEV), ccw_ref, slot)

            @pl.when(s < N_STEP)
            def _():
                rdma_cw.wait()

            @pl.when(s < N_STEP - 1)
            def _():
                rdma_ccw.wait()

            return carry

        lax.fori_loop(0, N_STEP + 1, step, 0)

    return pl.pallas_call(
        body,
        out_shape=jax.ShapeDtypeStruct((B_LOC, SQ, D_MODEL), jnp.float32),
        in_specs=[
            pl.BlockSpec(memory_space=pltpu.VMEM),
            pl.BlockSpec(memory_space=pltpu.VMEM),
            pl.BlockSpec(memory_space=pltpu.VMEM),
            pl.BlockSpec(memory_space=pltpu.VMEM),
        ],
        out_specs=pl.BlockSpec(memory_space=pltpu.VMEM),
        scratch_shapes=[
            pltpu.VMEM((2, 2, D_MODEL, DF_LOC), jnp.bfloat16),
            pltpu.VMEM((2, 2, D_MODEL, DF_LOC), jnp.bfloat16),
            pltpu.VMEM((TOK, DF_LOC), jnp.bfloat16),
            pltpu.SemaphoreType.DMA((2,)),
            pltpu.SemaphoreType.DMA((2,)),
            pltpu.SemaphoreType.DMA((2,)),
            pltpu.SemaphoreType.DMA((2,)),
        ],
        compiler_params=pltpu.CompilerParams(collective_id=0),
    )(x2d, wio, kh, vh)


# device time: 157414 ns/iter; 1.7058x vs baseline; 1.5391x over previous
import os

import jax
import jax.numpy as jnp
import numpy as np
from jax import lax
from jax.experimental import pallas as pl
from jax.experimental.pallas import tpu as pltpu

_DIAG = os.environ.get("KERNEL_DIAG", "")

N_DEV = 32
B_LOC = 2
SQ = 128
SKV = 128
HQ = 128
DH = 64
D_MODEL = 512
H_LOC = HQ // N_DEV
DF_LOC = H_LOC * DH
TOK = B_LOC * SQ
N_STEP = N_DEV // 2

_RING = [0, 8, 16, 24, 27, 19, 11, 3, 4, 12, 20, 28, 31, 23, 15, 7,
         6, 14, 22, 30, 29, 21, 13, 5, 2, 10, 18, 26, 25, 17, 9, 1]
_POS = [0] * N_DEV
for _p, _l in enumerate(_RING):
    _POS[_l] = _p
_NEXT = [_RING[(_POS[l] + 1) % N_DEV] for l in range(N_DEV)]
_PREV = [_RING[(_POS[l] - 1) % N_DEV] for l in range(N_DEV)]
_TBL = np.array([_NEXT, _PREV, _POS, _RING], dtype=np.int32)


def kernel(x, Wq, K_ext, V_ext, Wo):
    my_i = lax.axis_index("i")

    x2d = x.reshape(TOK, D_MODEL).astype(jnp.bfloat16)
    wio = jnp.stack([Wq.astype(jnp.bfloat16), Wo.T.astype(jnp.bfloat16)])

    k_loc = lax.dynamic_slice_in_dim(K_ext, my_i * B_LOC, B_LOC, axis=0)
    v_loc = lax.dynamic_slice_in_dim(V_ext, my_i * B_LOC, B_LOC, axis=0)
    kh = k_loc.transpose(2, 0, 1, 3).reshape(HQ * B_LOC, SKV, DH)
    vh = v_loc.transpose(2, 0, 1, 3).reshape(HQ * B_LOC, SKV, DH)
    kh = kh.astype(jnp.bfloat16)
    vh = vh.astype(jnp.bfloat16)

    tbl = jnp.asarray(_TBL)

    def body(tbl_ref, x_ref, wio_ref, k_ref, v_ref, out_ref,
             cw_ref, ccw_ref, ctx_ref,
             cw_send, cw_recv, ccw_send, ccw_recv):
        i = lax.axis_index("i")
        nxt = tbl_ref[0, i]
        prv = tbl_ref[1, i]
        pos = tbl_ref[2, i]

        barrier_sem = pltpu.get_barrier_semaphore()
        pl.semaphore_signal(barrier_sem, inc=1, device_id=(prv,),
                            device_id_type=pl.DeviceIdType.MESH)
        pl.semaphore_signal(barrier_sem, inc=1, device_id=(nxt,),
                            device_id_type=pl.DeviceIdType.MESH)
        pl.semaphore_wait(barrier_sem, 2)

        cw_ref[0] = wio_ref[...]
        ccw_ref[0] = wio_ref[...]
        out_ref[...] = jnp.zeros_like(out_ref)

        def compute_group(j, comm_ref, slot):
            wq = comm_ref[slot, 0]
            woT = comm_ref[slot, 1]

            q_all = lax.dot_general(
                x_ref[...], wq, (((1,), (0,)), ((), ())),
                preferred_element_type=jnp.float32)

            for b in range(B_LOC):
                for hh in range(H_LOC):
                    q = q_all[b * SQ:(b + 1) * SQ,
                              hh * DH:(hh + 1) * DH].astype(jnp.bfloat16)
                    idx = (j * H_LOC + hh) * B_LOC + b
                    k = k_ref[idx]
                    v = v_ref[idx]
                    s = lax.dot_general(
                        q, k, (((1,), (1,)), ((), ())),
                        preferred_element_type=jnp.float32) * 0.125
                    m = jnp.max(s, axis=1, keepdims=True)
                    e = jnp.exp(s - m)
                    w = (e / jnp.sum(e, axis=1, keepdims=True)
                         ).astype(jnp.bfloat16)
                    ctx = lax.dot_general(
                        w, v, (((1,), (0,)), ((), ())),
                        preferred_element_type=jnp.float32)
                    ctx_ref[b * SQ:(b + 1) * SQ,
                            hh * DH:(hh + 1) * DH] = ctx.astype(jnp.bfloat16)

            partial = lax.dot_general(
                ctx_ref[...], woT, (((1,), (1,)), ((), ())),
                preferred_element_type=jnp.float32)
            out_ref[...] += partial.reshape(B_LOC, SQ, D_MODEL)

        def step(s, carry):
            slot = lax.rem(s, 2)
            nslot = lax.rem(s + 1, 2)

            rdma_cw = pltpu.make_async_remote_copy(
                src_ref=cw_ref.at[slot],
                dst_ref=cw_ref.at[nslot],
                send_sem=cw_send.at[slot],
                recv_sem=cw_recv.at[nslot],
                device_id=(nxt,),
                device_id_type=pl.DeviceIdType.MESH,
            )
            rdma_ccw = pltpu.make_async_remote_copy(
                src_ref=ccw_ref.at[slot],
                dst_ref=ccw_ref.at[nslot],
                send_sem=ccw_send.at[slot],
                recv_sem=ccw_recv.at[nslot],
                device_id=(prv,),
                device_id_type=pl.DeviceIdType.MESH,
            )

            if _DIAG != "compute_only":
                @pl.when(s < N_STEP)
                def _():
                    rdma_cw.start()

                @pl.when(s < N_STEP - 1)
                def _():
                    rdma_ccw.start()

            if _DIAG != "comm_only":
                j_cw = tbl_ref[3, lax.rem(pos - s + N_DEV, N_DEV)]
                compute_group(j_cw, cw_ref, slot)

                @pl.when(jnp.logical_and(s >= 1, s < N_STEP))
                def _():
                    j_ccw = tbl_ref[3, lax.rem(pos + s, N_DEV)]
                    compute_group(j_ccw, ccw_ref, slot)

            if _DIAG != "compute_only":
                @pl.when(s < N_STEP)
                def _():
                    rdma_cw.wait()

                @pl.when(s < N_STEP - 1)
                def _():
                    rdma_ccw.wait()

            return carry

        lax.fori_loop(0, N_STEP + 1, step, 0)

    return pl.pallas_call(
        body,
        out_shape=jax.ShapeDtypeStruct((B_LOC, SQ, D_MODEL), jnp.float32),
        in_specs=[
            pl.BlockSpec(memory_space=pltpu.SMEM),
            pl.BlockSpec(memory_space=pltpu.VMEM),
            pl.BlockSpec(memory_space=pltpu.VMEM),
            pl.BlockSpec(memory_space=pltpu.VMEM),
            pl.BlockSpec(memory_space=pltpu.VMEM),
        ],
        out_specs=pl.BlockSpec(memory_space=pltpu.VMEM),
        scratch_shapes=[
            pltpu.VMEM((2, 2, D_MODEL, DF_LOC), jnp.bfloat16),
            pltpu.VMEM((2, 2, D_MODEL, DF_LOC), jnp.bfloat16),
            pltpu.VMEM((TOK, DF_LOC), jnp.bfloat16),
            pltpu.SemaphoreType.DMA((2,)),
            pltpu.SemaphoreType.DMA((2,)),
            pltpu.SemaphoreType.DMA((2,)),
            pltpu.SemaphoreType.DMA((2,)),
        ],
        compiler_params=pltpu.CompilerParams(collective_id=0),
    )(tbl, x2d, wio, kh, vh)
